# baseline (speedup 1.0000x reference)
import functools

import jax
import jax.numpy as jnp
from jax import lax
from jax.experimental import pallas as pl
from jax.experimental.pallas import tpu as pltpu

_TILE = 2048


def _router_body(x_ref, wt_ref, idx_ref, w_ref, *, n_experts):
    xt = x_ref[...]
    wt = wt_ref[...]
    logits = jnp.dot(xt, wt, preferred_element_type=jnp.float32)
    t = logits.shape[0]
    iota = lax.broadcasted_iota(jnp.int32, (t, n_experts), 1)
    m1 = jnp.max(logits, axis=-1, keepdims=True)
    i1 = jnp.min(jnp.where(logits == m1, iota, n_experts), axis=-1, keepdims=True)
    l2 = jnp.where(iota == i1, -jnp.inf, logits)
    m2 = jnp.max(l2, axis=-1, keepdims=True)
    i2 = jnp.min(jnp.where(l2 == m2, iota, n_experts), axis=-1, keepdims=True)
    r = jnp.exp(m2 - m1)
    w1 = 1.0 / (1.0 + r)
    idx_ref[...] = jnp.concatenate([i1, i2], axis=-1)
    w_ref[...] = jnp.concatenate([w1, r * w1], axis=-1)


def kernel(x, weight):
    n_experts, dim = weight.shape
    xf = x.reshape(-1, dim)
    n = xf.shape[0]
    grid = n // _TILE
    body = functools.partial(_router_body, n_experts=n_experts)
    idx, w = pl.pallas_call(
        body,
        grid=(grid,),
        in_specs=[
            pl.BlockSpec((_TILE, dim), lambda i: (i, 0)),
            pl.BlockSpec((dim, n_experts), lambda i: (0, 0)),
        ],
        out_specs=[
            pl.BlockSpec((_TILE, 2), lambda i: (i, 0)),
            pl.BlockSpec((_TILE, 2), lambda i: (i, 0)),
        ],
        out_shape=[
            jax.ShapeDtypeStruct((n, 2), jnp.int32),
            jax.ShapeDtypeStruct((n, 2), jnp.float32),
        ],
    )(xf, weight.T)
    return idx, w


# speedup vs baseline: 1.0489x; 1.0489x over previous
import functools

import jax
import jax.numpy as jnp
from jax import lax
from jax.experimental import pallas as pl

_TILE = 4096
_NEG_INF = float("-inf")


def _router_body(x_ref, wt_ref, idx_ref, w_ref, *, n_experts):
    xt = x_ref[...]
    wt = wt_ref[...]
    logits = jnp.dot(xt, wt, preferred_element_type=jnp.float32)
    e = n_experts
    tri = (lax.broadcasted_iota(jnp.int32, (e, e), 0)
           < lax.broadcasted_iota(jnp.int32, (e, e), 1)).astype(jnp.float32)
    col = lax.broadcasted_iota(jnp.int32, (e, 1), 0).astype(jnp.float32)

    def first_argmax(vals, m):
        eq = (vals == m).astype(jnp.float32)
        prefix = jnp.dot(eq, tri, preferred_element_type=jnp.float32)
        onehot = eq * (prefix == 0.0).astype(jnp.float32)
        idx = jnp.dot(onehot, col, preferred_element_type=jnp.float32)
        return onehot, idx

    m1 = jnp.max(logits, axis=-1, keepdims=True)
    oh1, i1 = first_argmax(logits, m1)
    l2 = jnp.where(oh1 > 0.0, _NEG_INF, logits)
    m2 = jnp.max(l2, axis=-1, keepdims=True)
    _, i2 = first_argmax(l2, m2)
    r = jnp.exp(m2 - m1)
    w1 = 1.0 / (1.0 + r)
    idx_ref[...] = jnp.concatenate([i1, i2], axis=-1).astype(jnp.int32)
    w_ref[...] = jnp.concatenate([w1, r * w1], axis=-1)


def kernel(x, weight):
    n_experts, dim = weight.shape
    xf = x.reshape(-1, dim)
    n = xf.shape[0]
    grid = n // _TILE
    body = functools.partial(_router_body, n_experts=n_experts)
    idx, w = pl.pallas_call(
        body,
        grid=(grid,),
        in_specs=[
            pl.BlockSpec((_TILE, dim), lambda i: (i, 0)),
            pl.BlockSpec((dim, n_experts), lambda i: (0, 0)),
        ],
        out_specs=[
            pl.BlockSpec((_TILE, 2), lambda i: (i, 0)),
            pl.BlockSpec((_TILE, 2), lambda i: (i, 0)),
        ],
        out_shape=[
            jax.ShapeDtypeStruct((n, 2), jnp.int32),
            jax.ShapeDtypeStruct((n, 2), jnp.float32),
        ],
    )(xf, weight.T)
    return idx, w
